# baseline (speedup 1.0000x reference)
import functools

import jax
import jax.numpy as jnp
from jax import lax
from jax.experimental import pallas as pl
from jax.experimental.pallas import tpu as pltpu
from jax.experimental.pallas import tpu_sc as plsc

_B = 1048576
_GRIDSZ = 256
_MID = 128


_ROWS = 8192
_RB = 1024


def _idx_body(o_ref, v_ref, out_ref):
    o = o_ref[...]
    v = v_ref[...]
    inf = jnp.max(jnp.abs(v), axis=0)
    a = v[0] / inf
    b = v[1] / inf
    c = v[2] / inf
    one = jnp.float32(1.0)
    face = jnp.zeros(a.shape, dtype=jnp.int32)
    for idx, cond in enumerate(
        (a >= one, a <= -one, b >= one, b <= -one, c >= one, c <= -one)
    ):
        face = jnp.where(cond, jnp.int32(idx), face)
    scale = jnp.float32(_GRIDSZ - 1)
    coords = jnp.clip((o / 2 + 0.5) * scale, 0.0, scale).astype(jnp.int32)
    i, j, k = coords[0], coords[1], coords[2]
    byte_idx = (((i << 8) | j) << 8 | k) * 6 + face
    out_ref[...] = byte_idx


def _compute_byte_idx(ot, vt):
    return pl.pallas_call(
        _idx_body,
        out_shape=jax.ShapeDtypeStruct((_ROWS, 128), jnp.int32),
        grid=(_ROWS // _RB,),
        in_specs=[
            pl.BlockSpec((3, _RB, 128), lambda r: (0, r, 0)),
            pl.BlockSpec((3, _RB, 128), lambda r: (0, r, 0)),
        ],
        out_specs=pl.BlockSpec((_RB, 128), lambda r: (r, 0)),
    )(ot, vt)


_NC = 2
_NS = 16
_NW = _NC * _NS
_WROWS = _ROWS // _NW
_KB = 16


def _gather_body(bidx_hbm, table_hbm, out_hbm, bidx_v, widx_v, words_v, sem):
    wid = lax.axis_index("s") * _NC + lax.axis_index("c")
    row0 = wid * _WROWS

    pltpu.sync_copy(bidx_hbm.at[pl.ds(row0, _WROWS), :], bidx_v)

    def comp(g, carry):
        for l in range(8):
            sl = pl.ds(l * 16, 16)
            widx_v[g, sl] = lax.shift_right_logical(bidx_v[g, sl], 2)
        return carry

    lax.fori_loop(0, _WROWS, comp, 0)

    def blk(bI, carry):
        def fire(g, c2):
            row = bI * _KB + g
            pltpu.make_async_copy(
                table_hbm.at[widx_v.at[row]], words_v.at[row], sem
            ).start()
            return c2

        lax.fori_loop(0, _KB, fire, 0)

        def drain(g, c2):
            row = bI * _KB + g
            pltpu.make_async_copy(
                table_hbm.at[widx_v.at[row]], words_v.at[row], sem
            ).wait()
            return c2

        lax.fori_loop(0, _KB, drain, 0)

        def extr(g, c2):
            row = bI * _KB + g
            for l in range(8):
                sl = pl.ds(l * 16, 16)
                bi = bidx_v[row, sl]
                w = words_v[row, sl]
                sh = lax.shift_left(jnp.bitwise_and(bi, 3), 3)
                val = jnp.bitwise_and(lax.shift_right_logical(w, sh), 255)
                words_v[row, sl] = jnp.where(
                    val > _MID, jnp.int32(1), jnp.int32(0)
                )
            return c2

        lax.fori_loop(0, _KB, extr, 0)
        return carry

    lax.fori_loop(0, _WROWS // _KB, blk, 0)

    pltpu.sync_copy(words_v, out_hbm.at[pl.ds(row0, _WROWS), :])


@functools.partial(jax.jit, static_argnums=())
def _sc_gather(bidx2, table):
    f = pl.kernel(
        _gather_body,
        out_type=jax.ShapeDtypeStruct((_ROWS, 128), jnp.int32),
        mesh=plsc.VectorSubcoreMesh(core_axis_name="c", subcore_axis_name="s"),
        scratch_types=[
            pltpu.VMEM((_WROWS, 128), jnp.int32),
            pltpu.VMEM((_WROWS, 128), jnp.int32),
            pltpu.VMEM((_WROWS, 128), jnp.int32),
            pltpu.SemaphoreType.DMA,
        ],
    )
    return f(bidx2, table)


def kernel(norm_ray_origins, viewdirs, cache):
    ot = norm_ray_origins.T.reshape(3, _ROWS, 128)
    vt = viewdirs.T.reshape(3, _ROWS, 128)
    byte_idx = _compute_byte_idx(ot, vt)
    table = jax.lax.bitcast_convert_type(cache.reshape(-1, 4), jnp.int32)
    out = _sc_gather(byte_idx, table)
    return out.reshape(_B).astype(bool)


# speedup vs baseline: 14.2428x; 14.2428x over previous
import functools

import jax
import jax.numpy as jnp
from jax import lax
from jax.experimental import pallas as pl
from jax.experimental.pallas import tpu as pltpu
from jax.experimental.pallas import tpu_sc as plsc

_B = 1048576
_GRIDSZ = 256
_MID = 128


_ROWS = 8192
_RB = 1024


def _idx_body(o_ref, v_ref, out_ref):
    o = o_ref[...]
    v = v_ref[...]
    inf = jnp.max(jnp.abs(v), axis=0)
    a = v[0] / inf
    b = v[1] / inf
    c = v[2] / inf
    one = jnp.float32(1.0)
    face = jnp.zeros(a.shape, dtype=jnp.int32)
    for idx, cond in enumerate(
        (a >= one, a <= -one, b >= one, b <= -one, c >= one, c <= -one)
    ):
        face = jnp.where(cond, jnp.int32(idx), face)
    scale = jnp.float32(_GRIDSZ - 1)
    coords = jnp.clip((o / 2 + 0.5) * scale, 0.0, scale).astype(jnp.int32)
    i, j, k = coords[0], coords[1], coords[2]
    phys = (
        ((i * 6 + face) << 16)
        | ((j >> 5) << 13)
        | ((k >> 7) << 12)
        | (((j >> 2) & 7) << 9)
        | ((k & 127) << 2)
        | (j & 3)
    )
    out_ref[...] = phys


def _compute_phys_idx(ot, vt):
    return pl.pallas_call(
        _idx_body,
        out_shape=jax.ShapeDtypeStruct((_ROWS, 128), jnp.int32),
        grid=(_ROWS // _RB,),
        in_specs=[
            pl.BlockSpec((3, _RB, 128), lambda r: (0, r, 0)),
            pl.BlockSpec((3, _RB, 128), lambda r: (0, r, 0)),
        ],
        out_specs=pl.BlockSpec((_RB, 128), lambda r: (r, 0)),
    )(ot, vt)


_NC = 2
_NS = 16
_NW = _NC * _NS
_WROWS = _ROWS // _NW
_KB = 16


_WN = _B // _NW


def _gather_body(bidx_hbm, table_hbm, out_hbm, bidx_v, widx_v, words_v, sem):
    wid = lax.axis_index("s") * _NC + lax.axis_index("c")
    base = wid * _WN

    pltpu.sync_copy(bidx_hbm.at[pl.ds(base, _WN)], bidx_v)

    def comp(g, carry):
        for l in range(8):
            sl = pl.ds(g * 128 + l * 16, 16)
            widx_v[sl] = lax.shift_right_logical(bidx_v[sl], 2)
        return carry

    lax.fori_loop(0, _WN // 128, comp, 0)

    def blk(bI, carry):
        def fire(g, c2):
            sl = pl.ds((bI * _KB + g) * 128, 128)
            pltpu.make_async_copy(
                table_hbm.at[widx_v.at[sl]], words_v.at[sl], sem
            ).start()
            return c2

        lax.fori_loop(0, _KB, fire, 0)

        def drain(g, c2):
            sl = pl.ds((bI * _KB + g) * 128, 128)
            pltpu.make_async_copy(
                table_hbm.at[widx_v.at[sl]], words_v.at[sl], sem
            ).wait()
            return c2

        lax.fori_loop(0, _KB, drain, 0)

        def extr(g, c2):
            for l in range(8):
                sl = pl.ds((bI * _KB + g) * 128 + l * 16, 16)
                bi = bidx_v[sl]
                w = words_v[sl]
                sh = lax.shift_left(jnp.bitwise_and(bi, 3), 3)
                val = jnp.bitwise_and(lax.shift_right_logical(w, sh), 255)
                words_v[sl] = jnp.where(
                    val > _MID, jnp.int32(1), jnp.int32(0)
                )
            return c2

        lax.fori_loop(0, _KB, extr, 0)
        return carry

    lax.fori_loop(0, _WN // (128 * _KB), blk, 0)

    pltpu.sync_copy(words_v, out_hbm.at[pl.ds(base, _WN)])


def _sc_gather(bidx_flat, table):
    f = pl.kernel(
        _gather_body,
        out_type=jax.ShapeDtypeStruct((_B,), jnp.int32),
        mesh=plsc.VectorSubcoreMesh(core_axis_name="c", subcore_axis_name="s"),
        scratch_types=[
            pltpu.VMEM((_WN,), jnp.int32),
            pltpu.VMEM((_WN,), jnp.int32),
            pltpu.VMEM((_WN,), jnp.int32),
            pltpu.SemaphoreType.DMA,
        ],
    )
    return f(bidx_flat, table)


def kernel(norm_ray_origins, viewdirs, cache):
    ot = norm_ray_origins.T.reshape(3, _ROWS, 128)
    vt = viewdirs.T.reshape(3, _ROWS, 128)
    phys_idx = _compute_phys_idx(ot, vt)
    table = jax.lax.bitcast_convert_type(
        cache.reshape(_GRIDSZ, 8, 8, 4, 2, 128, 6)
        .transpose(0, 6, 1, 4, 2, 5, 3),
        jnp.int32,
    ).reshape(_GRIDSZ * _GRIDSZ * _GRIDSZ * 6 // 4)
    out = _sc_gather(phys_idx.reshape(_B), table)
    return out.astype(bool)
